# baseline (speedup 1.0000x reference)
import functools

import jax
import jax.numpy as jnp
from jax import lax
from jax.experimental import pallas as pl
from jax.experimental.pallas import tpu as pltpu
from jax.experimental.pallas import tpu_sc as plsc

_G = 64
_D = 128
_R = 256
_SUB = _R // 128
_NE = 320000
_NP = 10240
_ECHUNKS = _NE // _R
_NCHUNKS = _NP // _R
_NW = 32
_NB = 3
_EFULL = (_ECHUNKS // _NW) // _NB * _NB


def _sc_body(ea_hbm, eb_hbm, x_hbm, b_hbm, m_hbm, z65_hbm,
             es_out, ns_out,
             d0, d1, d2, i0, i1, i2, bbuf, mbuf,
             sh_es, sh_ns,
             ld0, ld1, ld2, li0, li1, li2, sd0, sd1, sd2):
    c = lax.axis_index("c")
    s = lax.axis_index("s")
    wid = s * 2 + c

    @pl.when(s == 0)
    def _init():
        pltpu.sync_copy(z65_hbm, sh_es)
        pltpu.sync_copy(z65_hbm, sh_ns)

    plsc.subcore_barrier()

    bufs = ((d0, i0, ld0, li0, sd0),
            (d1, i1, ld1, li1, sd1),
            (d2, i2, ld2, li2, sd2))

    def _issue_load(t, d, i, ld, li):
        ci = wid + t * _NW
        pltpu.async_copy(ea_hbm.at[pl.ds(ci * _R, _R)], d, ld)
        pltpu.async_copy(eb_hbm.at[ci], i, li)

    def _wait_load(t, d, i, ld, li):
        ci = wid + t * _NW
        pltpu.make_async_copy(ea_hbm.at[pl.ds(ci * _R, _R)], d, ld).wait()
        pltpu.make_async_copy(eb_hbm.at[ci], i, li).wait()

    def _issue_scat(d, i, sdm):
        for j in range(_SUB):
            pltpu.async_copy(d.at[pl.ds(j * 128, 128)], sh_es.at[i.at[j]],
                             sdm, add=True)

    def _wait_scat(d, i, sdm):
        for j in range(_SUB):
            pltpu.make_async_copy(d.at[pl.ds(j * 128, 128)],
                                  sh_es.at[i.at[j]], sdm).wait()

    _issue_load(0, d0, i0, ld0, li0)
    _issue_load(1, d1, i1, ld1, li1)

    def edge_body(jj, carry):
        for b in range(_NB):
            t = jj * _NB + b
            d, i, ld, li, sdm = bufs[b]
            d2_, i2_, ld2_, li2_, sd2_ = bufs[(b + 2) % _NB]
            _wait_load(t, d, i, ld, li)

            @pl.when(t >= 1)
            def _():
                _wait_scat(d2_, i2_, sd2_)

            @pl.when(t + 2 < _EFULL)
            def _():
                _issue_load(t + 2, d2_, i2_, ld2_, li2_)

            _issue_scat(d, i, sdm)
        return carry

    lax.fori_loop(0, _EFULL // _NB, edge_body, 0)
    _wait_scat(*bufs[(_EFULL - 1) % _NB][0:2], bufs[(_EFULL - 1) % _NB][4])

    @pl.when(wid < _ECHUNKS - _EFULL * _NW)
    def _edge_tail():
        ci = _EFULL * _NW + wid
        pltpu.sync_copy(ea_hbm.at[pl.ds(ci * _R, _R)], d0)
        pltpu.sync_copy(eb_hbm.at[ci], i0)
        for j in range(_SUB):
            pltpu.sync_copy(d0.at[pl.ds(j * 128, 128)], sh_es.at[i0.at[j]],
                            add=True)

    n_n = (_NCHUNKS - wid + (_NW - 1)) // _NW

    def node_body(j, carry):
        ci = wid + j * _NW
        pltpu.sync_copy(x_hbm.at[pl.ds(ci * _R, _R)], d0)
        pltpu.sync_copy(b_hbm.at[ci], bbuf)
        pltpu.sync_copy(m_hbm.at[ci], mbuf)
        for r in range(_SUB):
            for k in range(8):
                b16 = bbuf[r, pl.ds(k * 16, 16)]
                m16 = mbuf[r, pl.ds(k * 16, 16)]
                i0[r, pl.ds(k * 16, 16)] = jnp.where(m16 != 0, b16, _G)
        for r in range(_SUB):
            pltpu.sync_copy(d0.at[pl.ds(r * 128, 128)], sh_ns.at[i0.at[r]],
                            add=True)
        return carry

    lax.fori_loop(0, n_n, node_body, 0)

    plsc.subcore_barrier()

    @pl.when(s == 0)
    def _writeback():
        pltpu.sync_copy(sh_es, es_out.at[c])
        pltpu.sync_copy(sh_ns, ns_out.at[c])


_sc_aggregate = functools.partial(
    pl.kernel,
    out_type=(
        jax.ShapeDtypeStruct((2, _G + 1, _D), jnp.float32),
        jax.ShapeDtypeStruct((2, _G + 1, _D), jnp.float32),
    ),
    mesh=plsc.VectorSubcoreMesh(core_axis_name="c", subcore_axis_name="s"),
    scratch_types=[
        pltpu.VMEM((_R, _D), jnp.float32),
        pltpu.VMEM((_R, _D), jnp.float32),
        pltpu.VMEM((_R, _D), jnp.float32),
        pltpu.VMEM((_SUB, 128), jnp.int32),
        pltpu.VMEM((_SUB, 128), jnp.int32),
        pltpu.VMEM((_SUB, 128), jnp.int32),
        pltpu.VMEM((_SUB, 128), jnp.int32),
        pltpu.VMEM((_SUB, 128), jnp.int32),
        pltpu.VMEM_SHARED((_G + 1, _D), jnp.float32),
        pltpu.VMEM_SHARED((_G + 1, _D), jnp.float32),
        pltpu.SemaphoreType.DMA,
        pltpu.SemaphoreType.DMA,
        pltpu.SemaphoreType.DMA,
        pltpu.SemaphoreType.DMA,
        pltpu.SemaphoreType.DMA,
        pltpu.SemaphoreType.DMA,
        pltpu.SemaphoreType.DMA,
        pltpu.SemaphoreType.DMA,
        pltpu.SemaphoreType.DMA,
    ],
)(_sc_body)


def _tc_counts_body(eb_ref, b_ref, m_ref, ec_ref, nc_ref):
    def body(g, carry):
        ebv = eb_ref[...].reshape(_NE // 128, 128)
        bm = jnp.where(m_ref[...] != 0, b_ref[...], -1).reshape(_NP // 128, 128)
        ec_ref[pl.ds(g, 1), :] = jnp.full(
            (1, 128), jnp.sum((ebv == g).astype(jnp.float32)))
        nc_ref[pl.ds(g, 1), :] = jnp.full(
            (1, 128), jnp.sum((bm == g).astype(jnp.float32)))
        return carry

    lax.fori_loop(0, _G, body, 0)


_tc_counts = pl.pallas_call(
    _tc_counts_body,
    out_shape=(jax.ShapeDtypeStruct((_G, 128), jnp.float32),
               jax.ShapeDtypeStruct((_G, 128), jnp.float32)),
)


def _tc_body(u_ref, es_ref, ns_ref, ec_ref, nc_ref,
             w1_ref, b1_ref, w2_ref, b2_ref, g_ref, be_ref, o_ref):
    es = (es_ref[0] + es_ref[1])[0:_G, :]
    ns = (ns_ref[0] + ns_ref[1])[0:_G, :]
    ecv = jnp.max(ec_ref[...], axis=1, keepdims=True)
    ncv = jnp.max(nc_ref[...], axis=1, keepdims=True)
    ea = es / jnp.maximum(ecv, 1.0)
    na = ns / jnp.maximum(ncv, 1.0)
    u = u_ref[...]
    hi = lax.Precision.HIGHEST
    h = (jnp.dot(u, w1_ref[0:_D, :], precision=hi)
         + jnp.dot(ea, w1_ref[_D:2 * _D, :], precision=hi)
         + jnp.dot(na, w1_ref[2 * _D:3 * _D, :], precision=hi)
         + b1_ref[...])
    h = jnp.maximum(h, 0.0)
    h2 = jnp.dot(h, w2_ref[...], precision=hi) + b2_ref[...]
    mu = jnp.mean(h2, axis=-1, keepdims=True)
    var = jnp.mean((h2 - mu) * (h2 - mu), axis=-1, keepdims=True)
    o_ref[...] = (h2 - mu) * lax.rsqrt(var + 1e-5) * g_ref[...] + be_ref[...]


_tc_mlp = pl.pallas_call(
    _tc_body,
    out_shape=jax.ShapeDtypeStruct((_G, _D), jnp.float32),
)


def kernel(u, edge_attr, x, batch, edge_batch, var_mask, W1, b1, W2, b2, gamma, beta):
    n = x.shape[0]
    xp = jnp.zeros((_NP, _D), jnp.float32).at[0:n].set(x)
    bp = jnp.full((_NP,), _G, jnp.int32).at[0:n].set(batch.astype(jnp.int32))
    mp = jnp.zeros((_NP,), jnp.int32).at[0:n].set(var_mask.astype(jnp.int32))
    z65 = jnp.zeros((_G + 1, _D), jnp.float32)
    eb3 = edge_batch.astype(jnp.int32).reshape(_ECHUNKS, _SUB, 128)
    bp3 = bp.reshape(_NCHUNKS, _SUB, 128)
    mp3 = mp.reshape(_NCHUNKS, _SUB, 128)

    es2, ns2 = _sc_aggregate(edge_attr, eb3, xp, bp3, mp3, z65)
    ecm, ncm = _tc_counts(eb3, bp3, mp3)

    return _tc_mlp(u, es2, ns2, ecm, ncm, W1,
                   b1.reshape(1, -1), W2, b2.reshape(1, -1),
                   gamma.reshape(1, -1), beta.reshape(1, -1))
